# baseline (device time: 32302 ns/iter reference)
import jax
import jax.numpy as jnp
from jax import lax
from jax.experimental import pallas as pl
from jax.experimental.pallas import tpu as pltpu

M = 1024
D = 1024
Q = M // 4


def kernel(partial, resid, gamma):
    p = partial.reshape(M, D)
    g = gamma.reshape(1, D)

    def body(
        p_ref,
        r_ref,
        g_ref,
        o_ref,
        rs_send_buf,
        rs_recv_buf,
        oq_buf,
        agy_buf,
        agx_buf,
        rs_send_sem,
        rs_recv_sem,
        ag_send_sems,
        agy_recv_sem,
        agx_recv_sems,
    ):
        my_x = lax.axis_index("x")
        my_y = lax.axis_index("y")
        y_nbr = (my_x, 1 - my_y)
        x_nbr = (1 - my_x, my_y)
        my_q = 2 * my_x + my_y
        yn_q = 2 * my_x + (1 - my_y)
        xn_q = 2 * (1 - my_x) + my_y
        dg_q = 2 * (1 - my_x) + (1 - my_y)

        barrier = pltpu.get_barrier_semaphore()
        for nbr in (y_nbr, x_nbr):
            pl.semaphore_signal(
                barrier, inc=1, device_id=nbr, device_id_type=pl.DeviceIdType.MESH
            )
        pl.semaphore_wait(barrier, 2)

        rs_send_buf[...] = p_ref[pl.ds(yn_q * Q, Q), :].astype(jnp.bfloat16)
        rs = pltpu.make_async_remote_copy(
            src_ref=rs_send_buf,
            dst_ref=rs_recv_buf,
            send_sem=rs_send_sem,
            recv_sem=rs_recv_sem,
            device_id=y_nbr,
            device_id_type=pl.DeviceIdType.MESH,
        )
        rs.start()

        rs.wait_recv()

        q_out = rs_recv_buf[...].astype(jnp.float32)
        o_ref[pl.ds(my_q * Q, Q), :] = q_out
        oq_buf[...] = rs_recv_buf[...]

        ag_y = pltpu.make_async_remote_copy(
            src_ref=oq_buf,
            dst_ref=agy_buf,
            send_sem=ag_send_sems.at[0],
            recv_sem=agy_recv_sem,
            device_id=y_nbr,
            device_id_type=pl.DeviceIdType.MESH,
        )
        ag_x = pltpu.make_async_remote_copy(
            src_ref=oq_buf,
            dst_ref=agx_buf.at[0],
            send_sem=ag_send_sems.at[1],
            recv_sem=agx_recv_sems.at[0],
            device_id=x_nbr,
            device_id_type=pl.DeviceIdType.MESH,
        )
        ag_y.start()
        ag_x.start()

        ag_y.wait_recv()
        fwd = pltpu.make_async_remote_copy(
            src_ref=agy_buf,
            dst_ref=agx_buf.at[1],
            send_sem=ag_send_sems.at[2],
            recv_sem=agx_recv_sems.at[1],
            device_id=x_nbr,
            device_id_type=pl.DeviceIdType.MESH,
        )
        fwd.start()
        o_ref[pl.ds(yn_q * Q, Q), :] = agy_buf[...].astype(jnp.float32)

        ag_x.wait_recv()
        o_ref[pl.ds(xn_q * Q, Q), :] = agx_buf[0].astype(jnp.float32)
        fwd.wait_recv()
        o_ref[pl.ds(dg_q * Q, Q), :] = agx_buf[1].astype(jnp.float32)

        rs.wait_send()
        ag_y.wait_send()
        ag_x.wait_send()
        fwd.wait_send()

    return pl.pallas_call(
        body,
        out_shape=jax.ShapeDtypeStruct((M, D), jnp.float32),
        in_specs=[pl.BlockSpec(memory_space=pltpu.VMEM)] * 3,
        out_specs=pl.BlockSpec(memory_space=pltpu.VMEM),
        scratch_shapes=[
            pltpu.VMEM((Q, D), jnp.bfloat16),
            pltpu.VMEM((Q, D), jnp.bfloat16),
            pltpu.VMEM((Q, D), jnp.bfloat16),
            pltpu.VMEM((Q, D), jnp.bfloat16),
            pltpu.VMEM((2, Q, D), jnp.bfloat16),
            pltpu.SemaphoreType.DMA,
            pltpu.SemaphoreType.DMA,
            pltpu.SemaphoreType.DMA((3,)),
            pltpu.SemaphoreType.DMA,
            pltpu.SemaphoreType.DMA((2,)),
        ],
        compiler_params=pltpu.CompilerParams(collective_id=0),
    )(p, resid, g)


# device time: 26324 ns/iter; 1.2271x vs baseline; 1.2271x over previous
import jax
import jax.numpy as jnp
from jax import lax
from jax.experimental import pallas as pl
from jax.experimental.pallas import tpu as pltpu

M = 1024
D = 1024
Q = M // 4
C = 4
CR = Q // C


def kernel(partial, resid, gamma):
    p = partial.reshape(M, D)
    g = gamma.reshape(1, D)

    def body(
        p_hbm,
        r_hbm,
        g_ref,
        o_ref,
        p_yn_buf,
        p_my_buf,
        r_my_buf,
        rs_send_buf,
        rs_recv_buf,
        oq_buf,
        agy_buf,
        agxo_buf,
        agf_buf,
        in_sems,
        rs_send_sems,
        rs_recv_sems,
        agy_send_sems,
        agy_recv_sems,
        agxo_send_sems,
        agxo_recv_sems,
        fwd_send_sems,
        fwd_recv_sems,
    ):
        my_x = lax.axis_index("x")
        my_y = lax.axis_index("y")
        y_nbr = (my_x, 1 - my_y)
        x_nbr = (1 - my_x, my_y)
        my_q = 2 * my_x + my_y
        yn_q = 2 * my_x + (1 - my_y)
        xn_q = 2 * (1 - my_x) + my_y
        dg_q = 2 * (1 - my_x) + (1 - my_y)

        barrier = pltpu.get_barrier_semaphore()
        for nbr in (y_nbr, x_nbr):
            pl.semaphore_signal(
                barrier, inc=1, device_id=nbr, device_id_type=pl.DeviceIdType.MESH
            )

        cp_pyn = pltpu.make_async_copy(
            p_hbm.at[pl.ds(yn_q * Q, Q)], p_yn_buf, in_sems.at[0]
        )
        cp_pmy = pltpu.make_async_copy(
            p_hbm.at[pl.ds(my_q * Q, Q)], p_my_buf, in_sems.at[1]
        )
        cp_rmy = pltpu.make_async_copy(
            r_hbm.at[pl.ds(my_q * Q, Q)], r_my_buf, in_sems.at[2]
        )
        cp_pyn.start()
        cp_pmy.start()
        cp_rmy.start()

        pl.semaphore_wait(barrier, 2)

        cp_pyn.wait()
        rs = []
        for c in range(C):
            rs_send_buf[c] = p_yn_buf[pl.ds(c * CR, CR), :].astype(jnp.bfloat16)
            r_ = pltpu.make_async_remote_copy(
                src_ref=rs_send_buf.at[c],
                dst_ref=rs_recv_buf.at[c],
                send_sem=rs_send_sems.at[c],
                recv_sem=rs_recv_sems.at[c],
                device_id=y_nbr,
                device_id_type=pl.DeviceIdType.MESH,
            )
            r_.start()
            rs.append(r_)

        cp_pmy.wait()
        cp_rmy.wait()

        ag_y, ag_x = [], []
        for c in range(C):
            rs[c].wait_recv()
            y = (
                p_my_buf[pl.ds(c * CR, CR), :]
                + r_my_buf[pl.ds(c * CR, CR), :]
                + rs_recv_buf[c].astype(jnp.float32)
            )
            ms = jnp.mean(y * y, axis=-1, keepdims=True) + 1e-6
            q_out = y * lax.rsqrt(ms) * g_ref[...]
            o_ref[pl.ds(my_q * Q + c * CR, CR), :] = q_out
            oq_buf[c] = q_out.astype(jnp.bfloat16)
            a_ = pltpu.make_async_remote_copy(
                src_ref=oq_buf.at[c],
                dst_ref=agy_buf.at[c],
                send_sem=agy_send_sems.at[c],
                recv_sem=agy_recv_sems.at[c],
                device_id=y_nbr,
                device_id_type=pl.DeviceIdType.MESH,
            )
            b_ = pltpu.make_async_remote_copy(
                src_ref=oq_buf.at[c],
                dst_ref=agxo_buf.at[c],
                send_sem=agxo_send_sems.at[c],
                recv_sem=agxo_recv_sems.at[c],
                device_id=x_nbr,
                device_id_type=pl.DeviceIdType.MESH,
            )
            a_.start()
            b_.start()
            ag_y.append(a_)
            ag_x.append(b_)

        fwd = []
        for c in range(C):
            ag_y[c].wait_recv()
            f_ = pltpu.make_async_remote_copy(
                src_ref=agy_buf.at[c],
                dst_ref=agf_buf.at[c],
                send_sem=fwd_send_sems.at[c],
                recv_sem=fwd_recv_sems.at[c],
                device_id=x_nbr,
                device_id_type=pl.DeviceIdType.MESH,
            )
            f_.start()
            fwd.append(f_)
            o_ref[pl.ds(yn_q * Q + c * CR, CR), :] = agy_buf[c].astype(jnp.float32)
            ag_x[c].wait_recv()
            o_ref[pl.ds(xn_q * Q + c * CR, CR), :] = agxo_buf[c].astype(jnp.float32)

        for c in range(C):
            fwd[c].wait_recv()
            o_ref[pl.ds(dg_q * Q + c * CR, CR), :] = agf_buf[c].astype(jnp.float32)

        for c in range(C):
            rs[c].wait_send()
            ag_y[c].wait_send()
            ag_x[c].wait_send()
            fwd[c].wait_send()

    return pl.pallas_call(
        body,
        out_shape=jax.ShapeDtypeStruct((M, D), jnp.float32),
        in_specs=[
            pl.BlockSpec(memory_space=pl.ANY),
            pl.BlockSpec(memory_space=pl.ANY),
            pl.BlockSpec(memory_space=pltpu.VMEM),
        ],
        out_specs=pl.BlockSpec(memory_space=pltpu.VMEM),
        scratch_shapes=[
            pltpu.VMEM((Q, D), jnp.float32),
            pltpu.VMEM((Q, D), jnp.float32),
            pltpu.VMEM((Q, D), jnp.float32),
            pltpu.VMEM((C, CR, D), jnp.bfloat16),
            pltpu.VMEM((C, CR, D), jnp.bfloat16),
            pltpu.VMEM((C, CR, D), jnp.bfloat16),
            pltpu.VMEM((C, CR, D), jnp.bfloat16),
            pltpu.VMEM((C, CR, D), jnp.bfloat16),
            pltpu.VMEM((C, CR, D), jnp.bfloat16),
            pltpu.SemaphoreType.DMA((3,)),
            pltpu.SemaphoreType.DMA((C,)),
            pltpu.SemaphoreType.DMA((C,)),
            pltpu.SemaphoreType.DMA((C,)),
            pltpu.SemaphoreType.DMA((C,)),
            pltpu.SemaphoreType.DMA((C,)),
            pltpu.SemaphoreType.DMA((C,)),
            pltpu.SemaphoreType.DMA((C,)),
            pltpu.SemaphoreType.DMA((C,)),
        ],
        compiler_params=pltpu.CompilerParams(collective_id=0),
    )(p, resid, g)
